# baseline (device time: 78916 ns/iter reference)
import jax
import jax.numpy as jnp
from jax import lax
from jax.experimental import pallas as pl
from jax.experimental.pallas import tpu as pltpu

N_DEV = 4
B_LOC = 2
SQ = 512
SKV = 512
H_LOC = 8
H_HALF = 4
DH = 64
D_MODEL = 768
BLK = 64
HD_HALF = H_HALF * DH


def _body(x_ref, wq_ref, k_ref, v_ref, wo_ref, out_ref,
          wqA, wqB, woA, woB,
          qA_s, qA_r, qB_s, qB_r, oA_s, oA_r, oB_s, oB_r):
    me = lax.axis_index("i")
    left = (me + N_DEV - 1) % N_DEV
    right = (me + 1) % N_DEV

    barrier_sem = pltpu.get_barrier_semaphore()
    for nbr in (left, right):
        pl.semaphore_signal(
            barrier_sem, inc=1,
            device_id=(nbr,), device_id_type=pl.DeviceIdType.MESH,
        )
    pl.semaphore_wait(barrier_sem, 2)

    wqA[0] = wq_ref[:, :HD_HALF]
    wqB[0] = wq_ref[:, HD_HALF:]
    woA[0] = wo_ref[:HD_HALF, :]
    woB[0] = wo_ref[HD_HALF:, :]

    qb = lax.broadcasted_iota(jnp.int32, (SQ, SKV), 0) // BLK
    kb = lax.broadcasted_iota(jnp.int32, (SQ, SKV), 1) // BLK
    keep = (qb == kb) | (kb == 0) | (((qb + kb) % 3) == 0)
    mask = jnp.where(keep, 0.0, -1e9).astype(jnp.float32)

    x = x_ref[...]

    def half_attn(wq_slot, wo_slot, head0):
        q = jnp.dot(x, wq_slot, preferred_element_type=jnp.float32)
        q = (q * 0.125).astype(jnp.bfloat16)
        ctx_rows = []
        for b in range(B_LOC):
            ctx_cols = []
            for hh in range(H_HALF):
                g = head0 + hh
                qh = q[b * SQ:(b + 1) * SQ, hh * DH:(hh + 1) * DH]
                kh = k_ref[g, b]
                sc = lax.dot_general(
                    qh, kh, (((1,), (1,)), ((), ())),
                    preferred_element_type=jnp.float32,
                )
                w = jnp.exp(sc + mask)
                d = jnp.sum(w, axis=1, keepdims=True)
                vh = v_ref[g, b]
                ctx = jnp.dot(w.astype(jnp.bfloat16), vh,
                              preferred_element_type=jnp.float32)
                ctx_cols.append((ctx * (1.0 / d)).astype(jnp.bfloat16))
            ctx_rows.append(jnp.concatenate(ctx_cols, axis=1))
        ctx = jnp.concatenate(ctx_rows, axis=0)
        return jnp.dot(ctx, wo_slot, preferred_element_type=jnp.float32)

    acc = jnp.zeros((B_LOC * SQ, D_MODEL), jnp.float32)
    for s in range(N_DEV):
        if s < N_DEV - 1:
            rdmas = []
            for buf, sems_s, sems_r, dst in (
                (wqA, qA_s, qA_r, right), (woA, oA_s, oA_r, right),
                (wqB, qB_s, qB_r, left), (woB, oB_s, oB_r, left),
            ):
                r = pltpu.make_async_remote_copy(
                    src_ref=buf.at[s], dst_ref=buf.at[s + 1],
                    send_sem=sems_s.at[s], recv_sem=sems_r.at[s],
                    device_id=(dst,), device_id_type=pl.DeviceIdType.MESH,
                )
                r.start()
                rdmas.append(r)

        acc = acc + half_attn(wqA[s], woA[s], s * H_LOC)
        acc = acc + half_attn(wqB[s], woB[s], s * H_LOC + H_HALF)

        if s < N_DEV - 1:
            for r in rdmas:
                r.wait()

    out_ref[...] = acc.astype(jnp.bfloat16)


def kernel(x, Wq, K_ext, V_ext, Wo):
    me = lax.axis_index("i")
    bf16 = jnp.bfloat16

    x2d = x.astype(bf16).reshape(B_LOC * SQ, D_MODEL)
    wq = Wq.astype(bf16)
    wo = Wo.astype(bf16)
    s_idx = jnp.arange(N_DEV)
    origin_r = (me - s_idx) % N_DEV
    origin_l = (me + s_idx) % N_DEV
    hh = jnp.arange(H_HALF)
    g_right = origin_r[:, None] * H_LOC + hh[None, :]
    g_left = origin_l[:, None] * H_LOC + H_HALF + hh[None, :]
    perm = jnp.concatenate([g_right, g_left], axis=1).reshape(32)
    kb = lax.dynamic_slice_in_dim(K_ext, me * B_LOC, B_LOC, axis=0)
    vb = lax.dynamic_slice_in_dim(V_ext, me * B_LOC, B_LOC, axis=0)
    kb = kb[:, :, perm, :].astype(bf16).transpose(2, 0, 1, 3)
    vb = vb[:, :, perm, :].astype(bf16).transpose(2, 0, 1, 3)

    dma3 = pltpu.SemaphoreType.DMA((N_DEV - 1,))
    out2d = pl.pallas_call(
        _body,
        out_shape=jax.ShapeDtypeStruct((B_LOC * SQ, D_MODEL), bf16),
        in_specs=[pl.BlockSpec(memory_space=pltpu.VMEM)] * 5,
        out_specs=pl.BlockSpec(memory_space=pltpu.VMEM),
        scratch_shapes=[
            pltpu.VMEM((N_DEV, D_MODEL, HD_HALF), bf16),
            pltpu.VMEM((N_DEV, D_MODEL, HD_HALF), bf16),
            pltpu.VMEM((N_DEV, HD_HALF, D_MODEL), bf16),
            pltpu.VMEM((N_DEV, HD_HALF, D_MODEL), bf16),
            dma3, dma3,
            dma3, dma3,
            dma3, dma3,
            dma3, dma3,
        ],
        compiler_params=pltpu.CompilerParams(
            collective_id=0, vmem_limit_bytes=64 * 1024 * 1024,
        ),
    )(x2d, wq, kb, vb, wo)

    return out2d.reshape(B_LOC, SQ, D_MODEL)


# device time: 74976 ns/iter; 1.0526x vs baseline; 1.0526x over previous
import jax
import jax.numpy as jnp
from jax import lax
from jax.experimental import pallas as pl
from jax.experimental.pallas import tpu as pltpu

N_DEV = 4
B_LOC = 2
SQ = 512
SKV = 512
H_LOC = 8
H_HALF = 4
DH = 64
D_MODEL = 768
BLK = 64
HD_HALF = H_HALF * DH


def _body(x_ref, wq_ref, k_ref, v_ref, wo_ref, out_ref,
          wqA, wqB, woA, woB,
          qA_s, qA_r, qB_s, qB_r, oA_s, oA_r, oB_s, oB_r):
    me = lax.axis_index("i")
    left = (me + N_DEV - 1) % N_DEV
    right = (me + 1) % N_DEV

    barrier_sem = pltpu.get_barrier_semaphore()
    for nbr in (left, right):
        pl.semaphore_signal(
            barrier_sem, inc=1,
            device_id=(nbr,), device_id_type=pl.DeviceIdType.MESH,
        )
    pl.semaphore_wait(barrier_sem, 2)

    wqA[0] = wq_ref[:, :HD_HALF]
    wqB[0] = wq_ref[:, HD_HALF:]
    woA[0] = wo_ref[:HD_HALF, :]
    woB[0] = wo_ref[HD_HALF:, :]

    qb = lax.broadcasted_iota(jnp.int32, (SQ, SKV), 0) // BLK
    kb = lax.broadcasted_iota(jnp.int32, (SQ, SKV), 1) // BLK
    keep = (qb == kb) | (kb == 0) | (((qb + kb) % 3) == 0)
    mask = jnp.where(keep, 0.0, -1e9).astype(jnp.float32)

    x = x_ref[...]

    def half_attn(wq_slot, wo_slot, head0):
        q = jnp.dot(x, wq_slot, preferred_element_type=jnp.float32)
        q = (q * 0.125).astype(jnp.bfloat16)
        ctx_rows = []
        for b in range(B_LOC):
            ctx_cols = []
            for hh in range(H_HALF):
                g = head0 + hh
                qh = q[b * SQ:(b + 1) * SQ, hh * DH:(hh + 1) * DH]
                kh = k_ref[g, b]
                sc = lax.dot_general(
                    qh, kh, (((1,), (1,)), ((), ())),
                    preferred_element_type=jnp.float32,
                )
                w = jnp.exp(sc + mask)
                d = jnp.sum(w, axis=1, keepdims=True)
                vh = v_ref[g, b]
                ctx = jnp.dot(w.astype(jnp.bfloat16), vh,
                              preferred_element_type=jnp.float32)
                ctx_cols.append((ctx * (1.0 / d)).astype(jnp.bfloat16))
            ctx_rows.append(jnp.concatenate(ctx_cols, axis=1))
        ctx = jnp.concatenate(ctx_rows, axis=0)
        return jnp.dot(ctx, wo_slot, preferred_element_type=jnp.float32)

    acc = jnp.zeros((B_LOC * SQ, D_MODEL), jnp.float32)
    for s in range(N_DEV):
        if s < N_DEV - 1:
            rdmas = []
            for buf, sems_s, sems_r, dst in (
                (wqA, qA_s, qA_r, right), (woA, oA_s, oA_r, right),
                (wqB, qB_s, qB_r, left), (woB, oB_s, oB_r, left),
            ):
                r = pltpu.make_async_remote_copy(
                    src_ref=buf.at[s], dst_ref=buf.at[s + 1],
                    send_sem=sems_s.at[s], recv_sem=sems_r.at[s],
                    device_id=(dst,), device_id_type=pl.DeviceIdType.MESH,
                )
                r.start()
                rdmas.append(r)

        origin_r = (me - s + N_DEV) % N_DEV
        origin_l = (me + s) % N_DEV
        acc = acc + half_attn(wqA[s], woA[s], origin_r * H_LOC)
        acc = acc + half_attn(wqB[s], woB[s], origin_l * H_LOC + H_HALF)

        if s < N_DEV - 1:
            for r in rdmas:
                r.wait()

    out_ref[...] = acc.astype(jnp.bfloat16)


def kernel(x, Wq, K_ext, V_ext, Wo):
    me = lax.axis_index("i")
    bf16 = jnp.bfloat16

    x2d = x.astype(bf16).reshape(B_LOC * SQ, D_MODEL)
    wq = Wq.astype(bf16)
    wo = Wo.astype(bf16)
    kb = lax.dynamic_slice_in_dim(K_ext, me * B_LOC, B_LOC, axis=0)
    vb = lax.dynamic_slice_in_dim(V_ext, me * B_LOC, B_LOC, axis=0)
    kb = kb.astype(bf16).transpose(2, 0, 1, 3)
    vb = vb.astype(bf16).transpose(2, 0, 1, 3)

    dma3 = pltpu.SemaphoreType.DMA((N_DEV - 1,))
    out2d = pl.pallas_call(
        _body,
        out_shape=jax.ShapeDtypeStruct((B_LOC * SQ, D_MODEL), bf16),
        in_specs=[pl.BlockSpec(memory_space=pltpu.VMEM)] * 5,
        out_specs=pl.BlockSpec(memory_space=pltpu.VMEM),
        scratch_shapes=[
            pltpu.VMEM((N_DEV, D_MODEL, HD_HALF), bf16),
            pltpu.VMEM((N_DEV, D_MODEL, HD_HALF), bf16),
            pltpu.VMEM((N_DEV, HD_HALF, D_MODEL), bf16),
            pltpu.VMEM((N_DEV, HD_HALF, D_MODEL), bf16),
            dma3, dma3,
            dma3, dma3,
            dma3, dma3,
            dma3, dma3,
        ],
        compiler_params=pltpu.CompilerParams(
            collective_id=0, vmem_limit_bytes=64 * 1024 * 1024,
        ),
    )(x2d, wq, kb, vb, wo)

    return out2d.reshape(B_LOC, SQ, D_MODEL)


# device time: 72219 ns/iter; 1.0927x vs baseline; 1.0382x over previous
import jax
import jax.numpy as jnp
from jax import lax
from jax.experimental import pallas as pl
from jax.experimental.pallas import tpu as pltpu

N_DEV = 4
B_LOC = 2
SQ = 512
SKV = 512
H_LOC = 8
H_HALF = 4
DH = 64
D_MODEL = 768
BLK = 64
HD_HALF = H_HALF * DH


def _body(x_ref, wq_ref, k_ref, v_ref, wo_ref, out_ref,
          wqA, wqB, woA, woB,
          qA_s, qA_r, qB_s, qB_r, oA_s, oA_r, oB_s, oB_r):
    me = lax.axis_index("i")
    left = (me + N_DEV - 1) % N_DEV
    right = (me + 1) % N_DEV

    barrier_sem = pltpu.get_barrier_semaphore()
    for nbr in (left, right):
        pl.semaphore_signal(
            barrier_sem, inc=1,
            device_id=(nbr,), device_id_type=pl.DeviceIdType.MESH,
        )
    pl.semaphore_wait(barrier_sem, 2)

    wqA[0] = wq_ref[:, :HD_HALF].astype(jnp.bfloat16)
    wqB[0] = wq_ref[:, HD_HALF:].astype(jnp.bfloat16)
    woA[0] = wo_ref[:HD_HALF, :].astype(jnp.bfloat16)
    woB[0] = wo_ref[HD_HALF:, :].astype(jnp.bfloat16)

    qb = lax.broadcasted_iota(jnp.int32, (SQ, SKV), 0) // BLK
    kb = lax.broadcasted_iota(jnp.int32, (SQ, SKV), 1) // BLK
    keep = (qb == kb) | (kb == 0) | (((qb + kb) % 3) == 0)
    mask = jnp.where(keep, 0.0, -1e9).astype(jnp.float32)

    x = x_ref[...].astype(jnp.bfloat16)

    def half_attn(wq_slot, wo_slot, head0):
        q = jnp.dot(x, wq_slot, preferred_element_type=jnp.float32)
        q = (q * 0.125).astype(jnp.bfloat16)
        ctx_rows = []
        for b in range(B_LOC):
            ctx_cols = []
            for hh in range(H_HALF):
                g = head0 + hh
                qh = q[b * SQ:(b + 1) * SQ, hh * DH:(hh + 1) * DH]
                kh = k_ref[g, b]
                sc = lax.dot_general(
                    qh, kh, (((1,), (1,)), ((), ())),
                    preferred_element_type=jnp.float32,
                )
                w = jnp.exp(sc + mask)
                d = jnp.sum(w, axis=1, keepdims=True)
                vh = v_ref[g, b]
                ctx = jnp.dot(w.astype(jnp.bfloat16), vh,
                              preferred_element_type=jnp.float32)
                ctx_cols.append((ctx * (1.0 / d)).astype(jnp.bfloat16))
            ctx_rows.append(jnp.concatenate(ctx_cols, axis=1))
        ctx = jnp.concatenate(ctx_rows, axis=0)
        return jnp.dot(ctx, wo_slot, preferred_element_type=jnp.float32)

    acc = jnp.zeros((B_LOC * SQ, D_MODEL), jnp.float32)
    for s in range(N_DEV):
        if s < N_DEV - 1:
            rdmas = []
            for buf, sems_s, sems_r, dst in (
                (wqA, qA_s, qA_r, right), (woA, oA_s, oA_r, right),
                (wqB, qB_s, qB_r, left), (woB, oB_s, oB_r, left),
            ):
                r = pltpu.make_async_remote_copy(
                    src_ref=buf.at[s], dst_ref=buf.at[s + 1],
                    send_sem=sems_s.at[s], recv_sem=sems_r.at[s],
                    device_id=(dst,), device_id_type=pl.DeviceIdType.MESH,
                )
                r.start()
                rdmas.append(r)

        origin_r = (me - s + N_DEV) % N_DEV
        origin_l = (me + s) % N_DEV
        acc = acc + half_attn(wqA[s], woA[s], origin_r * H_LOC)
        acc = acc + half_attn(wqB[s], woB[s], origin_l * H_LOC + H_HALF)

        if s < N_DEV - 1:
            for r in rdmas:
                r.wait()

    out_ref[...] = acc.astype(jnp.bfloat16)


def kernel(x, Wq, K_ext, V_ext, Wo):
    me = lax.axis_index("i")
    bf16 = jnp.bfloat16

    x2d = x.reshape(B_LOC * SQ, D_MODEL)
    wq = Wq
    wo = Wo
    kb = lax.dynamic_slice_in_dim(K_ext, me * B_LOC, B_LOC, axis=0)
    vb = lax.dynamic_slice_in_dim(V_ext, me * B_LOC, B_LOC, axis=0)
    kb = kb.astype(bf16).transpose(2, 0, 1, 3)
    vb = vb.astype(bf16).transpose(2, 0, 1, 3)

    dma3 = pltpu.SemaphoreType.DMA((N_DEV - 1,))
    out2d = pl.pallas_call(
        _body,
        out_shape=jax.ShapeDtypeStruct((B_LOC * SQ, D_MODEL), bf16),
        in_specs=[pl.BlockSpec(memory_space=pltpu.VMEM)] * 5,
        out_specs=pl.BlockSpec(memory_space=pltpu.VMEM),
        scratch_shapes=[
            pltpu.VMEM((N_DEV, D_MODEL, HD_HALF), bf16),
            pltpu.VMEM((N_DEV, D_MODEL, HD_HALF), bf16),
            pltpu.VMEM((N_DEV, HD_HALF, D_MODEL), bf16),
            pltpu.VMEM((N_DEV, HD_HALF, D_MODEL), bf16),
            dma3, dma3,
            dma3, dma3,
            dma3, dma3,
            dma3, dma3,
        ],
        compiler_params=pltpu.CompilerParams(
            collective_id=0, vmem_limit_bytes=64 * 1024 * 1024,
        ),
    )(x2d, wq, kb, vb, wo)

    return out2d.reshape(B_LOC, SQ, D_MODEL)
